# baseline (speedup 1.0000x reference)
import jax
import jax.numpy as jnp
from jax import lax
from jax.experimental import pallas as pl
from jax.experimental.pallas import tpu as pltpu
from jax.experimental.pallas import tpu_sc as plsc

B = 4096
PPAD = 104
DPAD = 128
XP = 128
NN = 50
NC = 50
NP = NN + NC
D = 64
LANES = 16

NW = 32
ROWS_PER_W = B // NW
CB = 8
NCHUNK = ROWS_PER_W // CB
WINS = (0, 16, 32, 34)


def _sc_body(x_hbm, off_hbm, tok_hbm, table_hbm, out_hbm,
             x0, x1, asm0, asm1, off_v, tok_v,
             sg0, sg1, so0, so1):
    xs, asms = (x0, x1), (asm0, asm1)
    sgs, sos = (sg0, sg1), (so0, so1)
    wid = lax.axis_index("s") * 2 + lax.axis_index("c")
    base_row = wid * ROWS_PER_W
    ivec = lax.iota(jnp.int32, LANES)

    pltpu.sync_copy(tok_hbm, tok_v)
    pltpu.sync_copy(off_hbm, off_v)

    def load_x(ci, buf):
        pltpu.sync_copy(
            x_hbm.at[pl.ds((base_row + ci * CB) * XP, CB * XP)], xs[buf])

    def fire_gathers(buf):
        for b in range(CB):
            ib = jnp.full((LANES,), b * XP, jnp.int32)
            for j0 in WINS:
                jv = ivec + j0
                codes = plsc.load_gather(xs[buf], [ib + (jv + NN)])
                offs = plsc.load_gather(off_v, [jv])
                iv = codes.astype(jnp.int32) + offs
                pltpu.async_copy(table_hbm.at[iv],
                                 asms[buf].at[b, pl.ds(NN + j0, LANES)],
                                 sgs[buf])

    def wait_gathers(buf):
        for b in range(CB):
            for j0 in WINS:
                pltpu.make_async_copy(
                    table_hbm.at[pl.ds(0, LANES)],
                    asms[buf].at[b, pl.ds(NN + j0, LANES)], sgs[buf]).wait()

    def noncat(buf):
        for p in range(NN):
            tokv = [tok_v[pl.ds(p * D + LANES * dd, LANES)]
                    for dd in range(D // LANES)]
            ip = jnp.full((LANES,), p, jnp.int32)

            @pl.loop(0, CB, unroll=4)
            def _ncb(b, tokv=tokv, ip=ip, buf=buf, p=p):
                sv = plsc.load_gather(
                    xs[buf], [jnp.full((LANES,), 0, jnp.int32) + b * XP + ip])
                for dd in range(D // LANES):
                    asms[buf][b, p, pl.ds(LANES * dd, LANES)] = \
                        tokv[dd] * sv

    def fire_out(ci, buf):
        pltpu.async_copy(
            asms[buf],
            out_hbm.at[pl.ds(base_row + ci * CB, CB), pl.ds(0, NP),
                       pl.ds(0, D)], sos[buf])

    def wait_out(buf):
        pltpu.make_async_copy(
            asms[buf],
            out_hbm.at[pl.ds(base_row, CB), pl.ds(0, NP), pl.ds(0, D)],
            sos[buf]).wait()

    load_x(0, 0)
    fire_gathers(0)

    @pl.loop(0, NCHUNK // 2)
    def _pair(ch):
        for sub in (0, 1):
            buf, nbuf = sub, 1 - sub
            ci = 2 * ch + sub
            noncat(buf)

            @pl.when(ci < NCHUNK - 1)
            def _pref(ci=ci, buf=buf, nbuf=nbuf):
                @pl.when(ci >= 1)
                def _drain():
                    wait_out(nbuf)
                load_x(ci + 1, nbuf)
                fire_gathers(nbuf)

            wait_gathers(buf)
            fire_out(ci, buf)

    wait_out(0)
    wait_out(1)


@jax.jit
def _tokenize(x, off, tok, table):
    mesh = plsc.VectorSubcoreMesh(core_axis_name="c", subcore_axis_name="s",
                                  num_cores=2, num_subcores=16)
    f = pl.kernel(
        _sc_body,
        out_type=jax.ShapeDtypeStruct((B, PPAD, DPAD), jnp.float32),
        mesh=mesh,
        scratch_types=[
            pltpu.VMEM((CB * XP,), jnp.float32),
            pltpu.VMEM((CB * XP,), jnp.float32),
            pltpu.VMEM((CB, NP, D), jnp.float32),
            pltpu.VMEM((CB, NP, D), jnp.float32),
            pltpu.VMEM((NC,), jnp.int32),
            pltpu.VMEM((NN * D,), jnp.float32),
            pltpu.SemaphoreType.DMA,
            pltpu.SemaphoreType.DMA,
            pltpu.SemaphoreType.DMA,
            pltpu.SemaphoreType.DMA,
        ],
        compiler_params=pltpu.CompilerParams(use_tc_tiling_on_sc=False,
                                             needs_layout_passes=False),
    )
    return f(x, off, tok, table)


def kernel(x, noncat_tokenizer, cat_table, noncat_idx, cat_idx, cat_offsets):
    xp = jnp.pad(x, ((0, 0), (0, XP - NP))).reshape(-1)
    out2 = _tokenize(xp, cat_offsets.astype(jnp.int32),
                     noncat_tokenizer.reshape(-1), cat_table)
    return out2[:, :NP, :D]


# speedup vs baseline: 1.0333x; 1.0333x over previous
import jax
import jax.numpy as jnp
from jax import lax
from jax.experimental import pallas as pl
from jax.experimental.pallas import tpu as pltpu
from jax.experimental.pallas import tpu_sc as plsc

B = 4096
PPAD = 104
DPAD = 128
XP = 128
NN = 50
NC = 50
NP = NN + NC
D = 64
LANES = 16

NW = 32
ROWS_PER_W = B // NW
CB = 8
NCHUNK = ROWS_PER_W // CB
WINS = (0, 16, 32, 34)


def _sc_body(x_hbm, off_hbm, tok_hbm, table_hbm, out_hbm,
             x0, x1, asm0, asm1, off_v, tok_v,
             sg0, sg1, so0, so1):
    xs, asms = (x0, x1), (asm0, asm1)
    sgs, sos = (sg0, sg1), (so0, so1)
    wid = lax.axis_index("s") * 2 + lax.axis_index("c")
    base_row = wid * ROWS_PER_W
    ivec = lax.iota(jnp.int32, LANES)

    pltpu.sync_copy(tok_hbm, tok_v)
    pltpu.sync_copy(off_hbm, off_v)

    def load_x(ci, buf):
        pltpu.sync_copy(x_hbm.at[pl.ds(base_row + ci * CB, CB)], xs[buf])

    def fire_gathers(buf):
        for b in range(CB):
            ib = jnp.full((LANES,), b, jnp.int32)
            for j0 in WINS:
                jv = ivec + j0
                codes = plsc.load_gather(xs[buf], [ib, jv + NN])
                offs = plsc.load_gather(off_v, [jv])
                iv = codes.astype(jnp.int32) + offs
                pltpu.async_copy(table_hbm.at[iv],
                                 asms[buf].at[b, pl.ds(NN + j0, LANES)],
                                 sgs[buf])

    def wait_gathers(buf):
        for b in range(CB):
            for j0 in WINS:
                pltpu.make_async_copy(
                    table_hbm.at[pl.ds(0, LANES)],
                    asms[buf].at[b, pl.ds(NN + j0, LANES)], sgs[buf]).wait()

    def noncat(buf):
        for p in range(NN):
            tokv = [tok_v[p, pl.ds(LANES * dd, LANES)]
                    for dd in range(D // LANES)]
            ip = jnp.full((LANES,), p, jnp.int32)

            @pl.loop(0, CB, unroll=4)
            def _ncb(b, tokv=tokv, ip=ip, buf=buf, p=p):
                ib = jnp.full((LANES,), 0, jnp.int32) + b
                sv = plsc.load_gather(xs[buf], [ib, ip])
                for dd in range(D // LANES):
                    asms[buf][b, p, pl.ds(LANES * dd, LANES)] = \
                        tokv[dd] * sv

    def fire_out(ci, buf):
        pltpu.async_copy(
            asms[buf],
            out_hbm.at[pl.ds(base_row + ci * CB, CB), pl.ds(0, NP),
                       pl.ds(0, D)], sos[buf])

    def wait_out(buf):
        pltpu.make_async_copy(
            asms[buf],
            out_hbm.at[pl.ds(base_row, CB), pl.ds(0, NP), pl.ds(0, D)],
            sos[buf]).wait()

    load_x(0, 0)
    fire_gathers(0)

    @pl.loop(0, NCHUNK // 2)
    def _pair(ch):
        for sub in (0, 1):
            buf, nbuf = sub, 1 - sub
            ci = 2 * ch + sub
            noncat(buf)

            @pl.when(ci < NCHUNK - 1)
            def _pref(ci=ci, buf=buf, nbuf=nbuf):
                @pl.when(ci >= 1)
                def _drain():
                    wait_out(nbuf)
                load_x(ci + 1, nbuf)
                fire_gathers(nbuf)

            wait_gathers(buf)
            fire_out(ci, buf)

    wait_out(0)
    wait_out(1)


@jax.jit
def _tokenize(x, off, tok, table):
    mesh = plsc.VectorSubcoreMesh(core_axis_name="c", subcore_axis_name="s",
                                  num_cores=2, num_subcores=16)
    f = pl.kernel(
        _sc_body,
        out_type=jax.ShapeDtypeStruct((B, PPAD, DPAD), jnp.float32),
        mesh=mesh,
        scratch_types=[
            pltpu.VMEM((CB, XP), jnp.float32),
            pltpu.VMEM((CB, XP), jnp.float32),
            pltpu.VMEM((CB, NP, D), jnp.float32),
            pltpu.VMEM((CB, NP, D), jnp.float32),
            pltpu.VMEM((NC,), jnp.int32),
            pltpu.VMEM((NN, D), jnp.float32),
            pltpu.SemaphoreType.DMA,
            pltpu.SemaphoreType.DMA,
            pltpu.SemaphoreType.DMA,
            pltpu.SemaphoreType.DMA,
        ],
        compiler_params=pltpu.CompilerParams(use_tc_tiling_on_sc=False,
                                             needs_layout_passes=False),
    )
    return f(x, off, tok, table)


def kernel(x, noncat_tokenizer, cat_table, noncat_idx, cat_idx, cat_offsets):
    xp = jnp.pad(x, ((0, 0), (0, XP - NP)))
    out2 = _tokenize(xp, cat_offsets.astype(jnp.int32), noncat_tokenizer,
                     cat_table)
    return out2[:, :NP, :D]
